# baseline (speedup 1.0000x reference)
import jax
import jax.numpy as jnp
from jax.experimental import pallas as pl

_MARGIN = 0.5
_EPS = 1e-6
_DIMS = (((1,), (1,)), ((), ()))


def _loss_kernel(x_ref, y_ref, lxy_ref, lyx_ref):
    x = x_ref[:]
    y = y_ref[:]
    B = x.shape[0]
    D = x.shape[1]

    z = x - y + _EPS
    zz = z * z
    ones_row = jnp.ones((1, D), dtype=jnp.float32)
    s_col = jnp.maximum(jnp.sum(zz, axis=1, keepdims=True), 1e-30)
    d_col = s_col * jax.lax.rsqrt(s_col)
    s_row = jnp.maximum(jax.lax.dot_general(
        ones_row, zz, _DIMS,
        preferred_element_type=jnp.float32), 1e-30)
    d_row = s_row * jax.lax.rsqrt(s_row)

    a = jnp.sum(x * (x + 2.0 * _EPS), axis=1, keepdims=True)
    b = jnp.sum(y * (y - 2.0 * _EPS), axis=1, keepdims=True) + D * _EPS * _EPS
    ones_col = jnp.ones((B, 1), dtype=jnp.float32)
    x_aug = jnp.concatenate([x, a, ones_col], axis=1)
    y_aug = jnp.concatenate([-2.0 * y, ones_col, b], axis=1)
    d2 = jax.lax.dot_general(
        x_aug, y_aug, _DIMS,
        preferred_element_type=jnp.float32)
    inv = 1.0 / (B * B)
    diag_corr = B * _MARGIN
    c_col = (_MARGIN + d_col).astype(jnp.bfloat16)
    c_row = (_MARGIN + d_row).astype(jnp.bfloat16)

    m = jnp.maximum(d2.astype(jnp.bfloat16), jnp.bfloat16(1e-30))
    dist = m * jax.lax.rsqrt(m)
    zero = jnp.bfloat16(0.0)
    lxy = jnp.maximum(c_col - dist, zero)
    lyx = jnp.maximum(c_row - dist, zero)
    ones_b = jnp.ones((1, B), dtype=jnp.bfloat16)
    red_dims = (((1,), (0,)), ((), ()))
    sxy = jax.lax.dot_general(ones_b, lxy, red_dims,
                              preferred_element_type=jnp.float32)
    syx = jax.lax.dot_general(ones_b, lyx, red_dims,
                              preferred_element_type=jnp.float32)
    lxy_ref[:, :] = (jnp.sum(sxy, axis=1, keepdims=True) - diag_corr) * inv
    lyx_ref[:, :] = (jnp.sum(syx, axis=1, keepdims=True) - diag_corr) * inv


def kernel(x_embed, y_embed):
    out_xy, out_yx = pl.pallas_call(
        _loss_kernel,
        out_shape=(
            jax.ShapeDtypeStruct((1, 1), jnp.float32),
            jax.ShapeDtypeStruct((1, 1), jnp.float32),
        ),
    )(x_embed, y_embed)
    return (out_xy[0, 0], out_yx[0, 0])
